# Initial kernel scaffold; baseline (speedup 1.0000x reference)
import functools

import jax
import jax.numpy as jnp
from jax import lax
from jax.experimental import pallas as pl
from jax.experimental.pallas import tpu as pltpu
from jax.experimental.pallas import tpu_sc as plsc

N_NODES = 10000
N_EDGES = 320000
D_EDGE = 16
D_FEAT = 128
D_GLOBAL = 32
H_DIM = 128

GRP = 128
NG = N_EDGES // GRP
NC = 2
NS = 16
GRP_PER_SC = NG // NC
GRP_BASE = GRP_PER_SC // NS
GRP_EXTRA = GRP_PER_SC - GRP_BASE * NS
W_GRPS = 13
N_WIN = GRP_BASE // W_GRPS
ROWS_PER_TILE = N_NODES // NS


def _sc_segment_sum(edges, idx2d):
  mesh = plsc.VectorSubcoreMesh(core_axis_name="c", subcore_axis_name="s")

  @functools.partial(
      pl.kernel,
      out_type=jax.ShapeDtypeStruct((NC, N_NODES, D_EDGE), jnp.float32),
      mesh=mesh,
      scratch_types=[
          pltpu.VMEM((W_GRPS, GRP), jnp.int32),
          pltpu.VMEM((W_GRPS * GRP, D_EDGE), jnp.float32),
          pltpu.VMEM((ROWS_PER_TILE, D_EDGE), jnp.float32),
          pltpu.VMEM_SHARED((N_NODES, D_EDGE), jnp.float32),
      ],
  )
  def seg_sum(edges_hbm, idx_hbm, out_hbm, idx_v, data_v, buf_v, acc_sh):
    c = lax.axis_index("c")
    s = lax.axis_index("s")

    zrow = jnp.zeros((D_EDGE,), jnp.float32)

    def zero_body(i, carry):
      buf_v[i] = zrow
      return carry

    lax.fori_loop(0, ROWS_PER_TILE, zero_body, 0)
    pltpu.sync_copy(buf_v, acc_sh.at[pl.ds(s * ROWS_PER_TILE, ROWS_PER_TILE)])
    plsc.subcore_barrier()

    base = c * GRP_PER_SC + s * GRP_BASE + jnp.minimum(s, GRP_EXTRA)

    def window(w, carry):
      g0 = base + w * W_GRPS
      pltpu.sync_copy(idx_hbm.at[pl.ds(g0, W_GRPS)], idx_v)
      pltpu.sync_copy(edges_hbm.at[pl.ds(g0 * GRP, W_GRPS * GRP)], data_v)
      for g in range(W_GRPS):
        pltpu.sync_copy(
            data_v.at[pl.ds(g * GRP, GRP)],
            acc_sh.at[idx_v.at[g]],
            add=True,
        )
      return carry

    lax.fori_loop(0, N_WIN, window, 0)

    @pl.when(s < GRP_EXTRA)
    def _extra():
      g0 = base + GRP_BASE
      pltpu.sync_copy(idx_hbm.at[pl.ds(g0, 1)], idx_v.at[pl.ds(0, 1)])
      pltpu.sync_copy(edges_hbm.at[pl.ds(g0 * GRP, GRP)],
                      data_v.at[pl.ds(0, GRP)])
      pltpu.sync_copy(data_v.at[pl.ds(0, GRP)], acc_sh.at[idx_v.at[0]],
                      add=True)

    plsc.subcore_barrier()

    pltpu.sync_copy(acc_sh.at[pl.ds(s * ROWS_PER_TILE, ROWS_PER_TILE)], buf_v)
    pltpu.sync_copy(buf_v,
                    out_hbm.at[c].at[pl.ds(s * ROWS_PER_TILE, ROWS_PER_TILE)])

  return seg_sum(edges, idx2d)


ROW_BLK = 1000


def _tc_mlp_ln(nodes, agg2, globals_, W1, b1, W2, b2, gamma, beta):
  grid = (N_NODES // ROW_BLK,)

  def body(nodes_ref, agg_ref, g_ref, w1_ref, b1_ref, w2_ref, b2_ref,
           gamma_ref, beta_ref, out_ref):
    agg = agg_ref[0] + agg_ref[1]
    w1n = w1_ref[:D_FEAT]
    w1g = w1_ref[D_FEAT:D_FEAT + D_GLOBAL]
    w1f = w1_ref[D_FEAT + D_GLOBAL:]
    bias1 = b1_ref[...] + jnp.dot(g_ref[...], w1g,
                                  preferred_element_type=jnp.float32)
    x = (jnp.dot(nodes_ref[...], w1n, preferred_element_type=jnp.float32)
         + jnp.dot(agg, w1f, preferred_element_type=jnp.float32)
         + bias1)
    h = jnp.maximum(x, 0.0)
    h = jnp.dot(h, w2_ref[...], preferred_element_type=jnp.float32)
    h = jnp.maximum(h + b2_ref[...], 0.0)
    mean = jnp.mean(h, axis=1, keepdims=True)
    d = h - mean
    var = jnp.mean(d * d, axis=1, keepdims=True)
    out_ref[...] = gamma_ref[...] * d * lax.rsqrt(var + 1e-3) + beta_ref[...]

  return pl.pallas_call(
      body,
      grid=grid,
      in_specs=[
          pl.BlockSpec((ROW_BLK, D_FEAT), lambda i: (i, 0)),
          pl.BlockSpec((NC, ROW_BLK, D_EDGE), lambda i: (0, i, 0)),
          pl.BlockSpec((1, D_GLOBAL), lambda i: (0, 0)),
          pl.BlockSpec((D_FEAT + D_GLOBAL + D_EDGE, H_DIM), lambda i: (0, 0)),
          pl.BlockSpec((1, H_DIM), lambda i: (0, 0)),
          pl.BlockSpec((H_DIM, H_DIM), lambda i: (0, 0)),
          pl.BlockSpec((1, H_DIM), lambda i: (0, 0)),
          pl.BlockSpec((1, H_DIM), lambda i: (0, 0)),
          pl.BlockSpec((1, H_DIM), lambda i: (0, 0)),
      ],
      out_specs=pl.BlockSpec((ROW_BLK, H_DIM), lambda i: (i, 0)),
      out_shape=jax.ShapeDtypeStruct((N_NODES, H_DIM), jnp.float32),
      compiler_params=pltpu.CompilerParams(
          dimension_semantics=("arbitrary",),
      ),
  )(nodes, agg2, globals_, W1, b1, W2, b2, gamma, beta)


@jax.jit
def kernel(nodes, globals_, n_node, hyperedges, hyperedge_index,
           W1, b1, W2, b2, gamma, beta):
  del n_node
  idx2d = hyperedge_index.reshape(NG, GRP)
  agg2 = _sc_segment_sum(hyperedges, idx2d)
  return _tc_mlp_ln(
      nodes, agg2, globals_, W1,
      b1.reshape(1, H_DIM), W2, b2.reshape(1, H_DIM),
      gamma.reshape(1, H_DIM), beta.reshape(1, H_DIM),
  )


# speedup vs baseline: 5.1306x; 5.1306x over previous
import functools

import jax
import jax.numpy as jnp
from jax import lax
from jax.experimental import pallas as pl
from jax.experimental.pallas import tpu as pltpu
from jax.experimental.pallas import tpu_sc as plsc

N_NODES = 10000
N_EDGES = 320000
D_EDGE = 16
D_FEAT = 128
D_GLOBAL = 32
H_DIM = 128

GRP = 128
NG = N_EDGES // GRP
NC = 2
NS = 16
GRP_PER_SC = NG // NC
GRP_BASE = GRP_PER_SC // NS
GRP_EXTRA = GRP_PER_SC - GRP_BASE * NS
W_GRPS = 13
N_WIN = GRP_BASE // W_GRPS
ROWS_PER_TILE = N_NODES // NS


def _sc_segment_sum(edges, idx2d):
  mesh = plsc.VectorSubcoreMesh(core_axis_name="c", subcore_axis_name="s")

  @functools.partial(
      pl.kernel,
      out_type=jax.ShapeDtypeStruct((NC, N_NODES, D_EDGE), jnp.float32),
      mesh=mesh,
      scratch_types=[
          pltpu.VMEM((W_GRPS, GRP), jnp.int32),
          pltpu.VMEM((W_GRPS * GRP, D_EDGE), jnp.float32),
          pltpu.VMEM((ROWS_PER_TILE, D_EDGE), jnp.float32),
          pltpu.VMEM_SHARED((N_NODES, D_EDGE), jnp.float32),
      ],
      compiler_params=pltpu.CompilerParams(use_tc_tiling_on_sc=False),
  )
  def seg_sum(edges_hbm, idx_hbm, out_hbm, idx_v, data_v, buf_v, acc_sh):
    c = lax.axis_index("c")
    s = lax.axis_index("s")

    zrow = jnp.zeros((D_EDGE,), jnp.float32)

    def zero_body(i, carry):
      buf_v[i] = zrow
      return carry

    lax.fori_loop(0, ROWS_PER_TILE, zero_body, 0)
    pltpu.sync_copy(buf_v, acc_sh.at[pl.ds(s * ROWS_PER_TILE, ROWS_PER_TILE)])
    plsc.subcore_barrier()

    base = c * GRP_PER_SC + s * GRP_BASE + jnp.minimum(s, GRP_EXTRA)

    def window(w, carry):
      g0 = base + w * W_GRPS
      pltpu.sync_copy(idx_hbm.at[pl.ds(g0, W_GRPS)], idx_v)
      pltpu.sync_copy(edges_hbm.at[pl.ds(g0 * GRP, W_GRPS * GRP)], data_v)
      for g in range(W_GRPS):
        pltpu.sync_copy(
            data_v.at[pl.ds(g * GRP, GRP)],
            acc_sh.at[idx_v.at[g]],
            add=True,
        )
      return carry

    lax.fori_loop(0, N_WIN, window, 0)

    @pl.when(s < GRP_EXTRA)
    def _extra():
      g0 = base + GRP_BASE
      pltpu.sync_copy(idx_hbm.at[pl.ds(g0, 1)], idx_v.at[pl.ds(0, 1)])
      pltpu.sync_copy(edges_hbm.at[pl.ds(g0 * GRP, GRP)],
                      data_v.at[pl.ds(0, GRP)])
      pltpu.sync_copy(data_v.at[pl.ds(0, GRP)], acc_sh.at[idx_v.at[0]],
                      add=True)

    plsc.subcore_barrier()

    pltpu.sync_copy(acc_sh.at[pl.ds(s * ROWS_PER_TILE, ROWS_PER_TILE)], buf_v)
    pltpu.sync_copy(buf_v,
                    out_hbm.at[c].at[pl.ds(s * ROWS_PER_TILE, ROWS_PER_TILE)])

  return seg_sum(edges, idx2d)


ROW_BLK = 1000


def _tc_mlp_ln(nodes, agg2, globals_, W1, b1, W2, b2, gamma, beta):
  grid = (N_NODES // ROW_BLK,)

  def body(nodes_ref, agg_ref, g_ref, w1_ref, b1_ref, w2_ref, b2_ref,
           gamma_ref, beta_ref, out_ref):
    agg = agg_ref[0] + agg_ref[1]
    w1n = w1_ref[:D_FEAT]
    w1g = w1_ref[D_FEAT:D_FEAT + D_GLOBAL]
    w1f = w1_ref[D_FEAT + D_GLOBAL:]
    bias1 = b1_ref[...] + jnp.dot(g_ref[...], w1g,
                                  preferred_element_type=jnp.float32)
    x = (jnp.dot(nodes_ref[...], w1n, preferred_element_type=jnp.float32)
         + jnp.dot(agg, w1f, preferred_element_type=jnp.float32)
         + bias1)
    h = jnp.maximum(x, 0.0)
    h = jnp.dot(h, w2_ref[...], preferred_element_type=jnp.float32)
    h = jnp.maximum(h + b2_ref[...], 0.0)
    mean = jnp.mean(h, axis=1, keepdims=True)
    d = h - mean
    var = jnp.mean(d * d, axis=1, keepdims=True)
    out_ref[...] = gamma_ref[...] * d * lax.rsqrt(var + 1e-3) + beta_ref[...]

  return pl.pallas_call(
      body,
      grid=grid,
      in_specs=[
          pl.BlockSpec((ROW_BLK, D_FEAT), lambda i: (i, 0)),
          pl.BlockSpec((NC, ROW_BLK, D_EDGE), lambda i: (0, i, 0)),
          pl.BlockSpec((1, D_GLOBAL), lambda i: (0, 0)),
          pl.BlockSpec((D_FEAT + D_GLOBAL + D_EDGE, H_DIM), lambda i: (0, 0)),
          pl.BlockSpec((1, H_DIM), lambda i: (0, 0)),
          pl.BlockSpec((H_DIM, H_DIM), lambda i: (0, 0)),
          pl.BlockSpec((1, H_DIM), lambda i: (0, 0)),
          pl.BlockSpec((1, H_DIM), lambda i: (0, 0)),
          pl.BlockSpec((1, H_DIM), lambda i: (0, 0)),
      ],
      out_specs=pl.BlockSpec((ROW_BLK, H_DIM), lambda i: (i, 0)),
      out_shape=jax.ShapeDtypeStruct((N_NODES, H_DIM), jnp.float32),
      compiler_params=pltpu.CompilerParams(
          dimension_semantics=("arbitrary",),
      ),
  )(nodes, agg2, globals_, W1, b1, W2, b2, gamma, beta)


@jax.jit
def kernel(nodes, globals_, n_node, hyperedges, hyperedge_index,
           W1, b1, W2, b2, gamma, beta):
  del n_node
  idx2d = hyperedge_index.reshape(NG, GRP)
  agg2 = _sc_segment_sum(hyperedges, idx2d)
  return _tc_mlp_ln(
      nodes, agg2, globals_, W1,
      b1.reshape(1, H_DIM), W2, b2.reshape(1, H_DIM),
      gamma.reshape(1, H_DIM), beta.reshape(1, H_DIM),
  )
